# baseline (speedup 1.0000x reference)
import functools

import jax
import jax.numpy as jnp
from jax import lax
from jax.experimental import pallas as pl
from jax.experimental.pallas import tpu as pltpu

H = 16
L = 32
N = L * L


def _tc_body(b0_ref, b1_ref, out_ref):
    h = pl.program_id(0)
    i = lax.broadcasted_iota(jnp.int32, (N, N), 0)
    j = lax.broadcasted_iota(jnp.int32, (N, N), 1)
    idx0 = 32 + (j >> 5) - (i >> 5)
    idx1 = 32 + (j & 31) - (i & 31)
    b0h = jnp.broadcast_to(b0_ref[h], (N, 2 * L))
    b1h = jnp.broadcast_to(b1_ref[h], (N, 2 * L))
    e0 = jnp.take_along_axis(b0h, idx0, axis=1)
    e1 = jnp.take_along_axis(b1h, idx1, axis=1)
    out_ref[0] = e0 + e1


def kernel(bias_0, bias_1):
    return pl.pallas_call(
        _tc_body,
        out_shape=jax.ShapeDtypeStruct((H, N, N), jnp.float32),
        grid=(H,),
        in_specs=[
            pl.BlockSpec((H, 2 * L), lambda h: (0, 0)),
            pl.BlockSpec((H, 2 * L), lambda h: (0, 0)),
        ],
        out_specs=pl.BlockSpec((1, N, N), lambda h: (h, 0, 0)),
    )(bias_0, bias_1)


# speedup vs baseline: 4.1010x; 4.1010x over previous
import functools

import jax
import jax.numpy as jnp
from jax import lax
from jax.experimental import pallas as pl
from jax.experimental.pallas import tpu as pltpu

H = 16
L = 32
N = L * L


def _tc_body(b0_ref, b1_ref, out_ref):
    h = pl.program_id(0)
    r = lax.broadcasted_iota(jnp.int32, (L, N), 0)
    j = lax.broadcasted_iota(jnp.int32, (L, N), 1)
    idx0 = 32 + (j >> 5) - r
    idx1 = 32 + (j & 31) - r
    b0h = jnp.broadcast_to(b0_ref[h], (L, 2 * L))
    b1h = jnp.broadcast_to(b1_ref[h], (L, 2 * L))
    e0 = jnp.take_along_axis(b0h, idx0, axis=1)
    e1 = jnp.take_along_axis(b1h, idx1, axis=1)
    out3 = e0[:, None, :] + e1[None, :, :]
    out_ref[0] = out3.reshape(N, N)


def kernel(bias_0, bias_1):
    return pl.pallas_call(
        _tc_body,
        out_shape=jax.ShapeDtypeStruct((H, N, N), jnp.float32),
        grid=(H,),
        in_specs=[
            pl.BlockSpec((H, 2 * L), lambda h: (0, 0)),
            pl.BlockSpec((H, 2 * L), lambda h: (0, 0)),
        ],
        out_specs=pl.BlockSpec((1, N, N), lambda h: (h, 0, 0)),
    )(bias_0, bias_1)
